# Initial kernel scaffold; baseline (speedup 1.0000x reference)
import jax
import jax.numpy as jnp
from jax.experimental import pallas as pl


def kernel(x, table):
    raise NotImplementedError("write your pallas kernel here")


# speedup vs baseline: 3.4072x; 3.4072x over previous
import functools

import jax
import jax.numpy as jnp
from jax import lax
from jax.experimental import pallas as pl
from jax.experimental.pallas import tpu as pltpu
from jax.experimental.pallas import tpu_sc as plsc

EMBED_DIM = 128
NUM_WORKERS = 32
CHUNK = 800


@jax.jit
def _embed(idx_flat, table):
    n = idx_flat.shape[0]
    per_w = n // NUM_WORKERS
    nchunks = per_w // CHUNK
    mesh = plsc.VectorSubcoreMesh(core_axis_name="c", subcore_axis_name="s")

    @functools.partial(
        pl.kernel,
        mesh=mesh,
        out_type=jax.ShapeDtypeStruct((n, EMBED_DIM), jnp.float32),
        scratch_types=[
            pltpu.VMEM((CHUNK,), jnp.int32),
            pltpu.VMEM((CHUNK, EMBED_DIM), jnp.float32),
            pltpu.SemaphoreType.DMA,
        ],
    )
    def k(table_hbm, idx_hbm, out_hbm, idx_v, rows_v, sem):
        wid = lax.axis_index("s") * 2 + lax.axis_index("c")
        base = wid * per_w

        @pl.loop(0, nchunks)
        def _(i):
            off = base + i * CHUNK
            pltpu.sync_copy(idx_hbm.at[pl.ds(off, CHUNK)], idx_v)
            pltpu.async_copy(table_hbm.at[idx_v], rows_v, sem).wait()
            pltpu.sync_copy(rows_v, out_hbm.at[pl.ds(off, CHUNK)])

    return k(table, idx_flat)


def kernel(x, table):
    b, s = x.shape
    out = _embed(x.reshape(-1), table)
    return out.reshape(b, s, EMBED_DIM)


# speedup vs baseline: 3.4595x; 1.0153x over previous
import functools

import jax
import jax.numpy as jnp
from jax import lax
from jax.experimental import pallas as pl
from jax.experimental.pallas import tpu as pltpu
from jax.experimental.pallas import tpu_sc as plsc

EMBED_DIM = 128
NUM_WORKERS = 32
CHUNK = 400
NBUF = 2


@jax.jit
def _embed(idx_flat, table):
    n = idx_flat.shape[0]
    per_w = n // NUM_WORKERS
    nchunks = per_w // CHUNK
    mesh = plsc.VectorSubcoreMesh(core_axis_name="c", subcore_axis_name="s")

    @functools.partial(
        pl.kernel,
        mesh=mesh,
        out_type=jax.ShapeDtypeStruct((n, EMBED_DIM), jnp.float32),
        scratch_types=[
            pltpu.VMEM((CHUNK,), jnp.int32),
            pltpu.VMEM((CHUNK,), jnp.int32),
            pltpu.VMEM((CHUNK, EMBED_DIM), jnp.float32),
            pltpu.VMEM((CHUNK, EMBED_DIM), jnp.float32),
            pltpu.SemaphoreType.DMA,
            pltpu.SemaphoreType.DMA,
            pltpu.SemaphoreType.DMA,
            pltpu.SemaphoreType.DMA,
        ],
    )
    def k(table_hbm, idx_hbm, out_hbm, idx0, idx1, rows0, rows1,
          gsem0, gsem1, ssem0, ssem1):
        idxb = (idx0, idx1)
        rows = (rows0, rows1)
        gsem = (gsem0, gsem1)
        ssem = (ssem0, ssem1)
        wid = lax.axis_index("s") * 2 + lax.axis_index("c")
        base = wid * per_w

        def load_idx(c, b):
            pltpu.sync_copy(idx_hbm.at[pl.ds(base + c * CHUNK, CHUNK)],
                            idxb[b])

        def gather(b):
            return pltpu.make_async_copy(
                table_hbm.at[idxb[b]], rows[b], gsem[b])

        def store(c, b):
            return pltpu.make_async_copy(
                rows[b], out_hbm.at[pl.ds(base + c * CHUNK, CHUNK)], ssem[b])

        for b in range(NBUF):
            load_idx(b, b)
            gather(b).start()

        @pl.loop(0, nchunks - NBUF, step=NBUF)
        def _(g):
            for b in range(NBUF):
                c = g + b
                gather(b).wait()
                store(c, b).start()
                load_idx(c + NBUF, b)
                store(c, b).wait()
                gather(b).start()

        for b in range(NBUF):
            c = nchunks - NBUF + b
            gather(b).wait()
            store(c, b).start()
        for b in range(NBUF):
            store(nchunks - NBUF + b, b).wait()

    return k(table, idx_flat)


def kernel(x, table):
    b, s = x.shape
    out = _embed(x.reshape(-1), table)
    return out.reshape(b, s, EMBED_DIM)


# speedup vs baseline: 6.2792x; 1.8151x over previous
import functools

import jax
import jax.numpy as jnp
from jax import lax
from jax.experimental import pallas as pl
from jax.experimental.pallas import tpu as pltpu
from jax.experimental.pallas import tpu_sc as plsc

EMBED_DIM = 128
NUM_WORKERS = 32
ROWS_PER_B = 50
B_PER_CHUNK = 8
CHUNK = B_PER_CHUNK * ROWS_PER_B
NBUF = 2


@functools.partial(jax.jit, static_argnums=2)
def _embed(idx_flat, table, batch):
    n = idx_flat.shape[0]
    per_w = n // NUM_WORKERS
    b_per_w = batch // NUM_WORKERS
    nchunks = b_per_w // B_PER_CHUNK
    mesh = plsc.VectorSubcoreMesh(core_axis_name="c", subcore_axis_name="s")

    @functools.partial(
        pl.kernel,
        mesh=mesh,
        out_type=jax.ShapeDtypeStruct((batch, ROWS_PER_B, EMBED_DIM),
                                      jnp.float32),
        compiler_params=pltpu.CompilerParams(use_tc_tiling_on_sc=True),
        scratch_types=[
            pltpu.VMEM((CHUNK,), jnp.int32),
            pltpu.VMEM((CHUNK,), jnp.int32),
            pltpu.VMEM((CHUNK, EMBED_DIM), jnp.float32),
            pltpu.VMEM((CHUNK, EMBED_DIM), jnp.float32),
            pltpu.SemaphoreType.DMA,
            pltpu.SemaphoreType.DMA,
            pltpu.SemaphoreType.DMA,
            pltpu.SemaphoreType.DMA,
        ],
    )
    def k(table_hbm, idx_hbm, out_hbm, idx0, idx1, rows0, rows1,
          gsem0, gsem1, ssem0, ssem1):
        idxb = (idx0, idx1)
        rows = (rows0, rows1)
        gsem = (gsem0, gsem1)
        ssem = (ssem0, ssem1)
        wid = lax.axis_index("s") * 2 + lax.axis_index("c")
        base = wid * per_w
        bbase = wid * b_per_w

        def load_idx(c, b):
            pltpu.sync_copy(idx_hbm.at[pl.ds(base + c * CHUNK, CHUNK)],
                            idxb[b])

        def gather(b):
            return pltpu.make_async_copy(
                table_hbm.at[idxb[b]], rows[b], gsem[b])

        def stores(c, b):
            return [
                pltpu.make_async_copy(
                    rows[b].at[pl.ds(j * ROWS_PER_B, ROWS_PER_B)],
                    out_hbm.at[bbase + c * B_PER_CHUNK + j],
                    ssem[b])
                for j in range(B_PER_CHUNK)
            ]

        for b in range(NBUF):
            load_idx(b, b)
            gather(b).start()

        @pl.loop(0, nchunks - NBUF, step=NBUF)
        def _(g):
            for b in range(NBUF):
                c = g + b
                gather(b).wait()
                for st in stores(c, b):
                    st.start()
                load_idx(c + NBUF, b)
                for st in stores(c, b):
                    st.wait()
                gather(b).start()

        for b in range(NBUF):
            c = nchunks - NBUF + b
            gather(b).wait()
            for st in stores(c, b):
                st.start()
        for b in range(NBUF):
            for st in stores(nchunks - NBUF + b, b):
                st.wait()

    return k(table, idx_flat)


def kernel(x, table):
    b, s = x.shape
    return _embed(x.reshape(-1), table, b)


# speedup vs baseline: 11.9569x; 1.9042x over previous
import functools

import jax
import jax.numpy as jnp
from jax import lax
from jax.experimental import pallas as pl
from jax.experimental.pallas import tpu as pltpu
from jax.experimental.pallas import tpu_sc as plsc

EMBED_DIM = 128
NUM_WORKERS = 32
CHUNK = 400
NBUF = 2


def _embed(idx_flat, table):
    n = idx_flat.shape[0]
    per_w = n // NUM_WORKERS
    nchunks = per_w // CHUNK
    mesh = plsc.VectorSubcoreMesh(core_axis_name="c", subcore_axis_name="s")

    @functools.partial(
        pl.kernel,
        mesh=mesh,
        out_type=jax.ShapeDtypeStruct((n, EMBED_DIM), jnp.float32),
        compiler_params=pltpu.CompilerParams(use_tc_tiling_on_sc=True),
        scratch_types=[
            pltpu.VMEM((CHUNK,), jnp.int32),
            pltpu.VMEM((CHUNK,), jnp.int32),
            pltpu.VMEM((CHUNK, EMBED_DIM), jnp.float32),
            pltpu.VMEM((CHUNK, EMBED_DIM), jnp.float32),
            pltpu.SemaphoreType.DMA,
            pltpu.SemaphoreType.DMA,
            pltpu.SemaphoreType.DMA,
            pltpu.SemaphoreType.DMA,
        ],
    )
    def k(table_hbm, idx_hbm, out_hbm, idx0, idx1, rows0, rows1,
          gsem0, gsem1, ssem0, ssem1):
        idxb = (idx0, idx1)
        rows = (rows0, rows1)
        gsem = (gsem0, gsem1)
        ssem = (ssem0, ssem1)
        wid = lax.axis_index("s") * 2 + lax.axis_index("c")
        base = wid * per_w

        def load_idx(c, b):
            pltpu.sync_copy(idx_hbm.at[pl.ds(base + c * CHUNK, CHUNK)],
                            idxb[b])

        def gather(b):
            return pltpu.make_async_copy(
                table_hbm.at[idxb[b]], rows[b], gsem[b])

        def store(c, b):
            return pltpu.make_async_copy(
                rows[b], out_hbm.at[pl.ds(base + c * CHUNK, CHUNK)], ssem[b])

        for b in range(NBUF):
            load_idx(b, b)
            gather(b).start()

        @pl.loop(0, nchunks - NBUF, step=NBUF)
        def _(g):
            for b in range(NBUF):
                c = g + b
                gather(b).wait()
                store(c, b).start()
                load_idx(c + NBUF, b)
                store(c, b).wait()
                gather(b).start()

        for b in range(NBUF):
            c = nchunks - NBUF + b
            gather(b).wait()
            store(c, b).start()
        for b in range(NBUF):
            store(nchunks - NBUF + b, b).wait()

    return k(table, idx_flat)


def kernel(x, table):
    b, s = x.shape
    out = _embed(x.T.reshape(-1), table)
    return out.reshape(s, b, EMBED_DIM).transpose(1, 0, 2)


# speedup vs baseline: 11.9772x; 1.0017x over previous
import functools

import jax
import jax.numpy as jnp
from jax import lax
from jax.experimental import pallas as pl
from jax.experimental.pallas import tpu as pltpu
from jax.experimental.pallas import tpu_sc as plsc

EMBED_DIM = 128
NUM_WORKERS = 32
CHUNK = 200
NBUF = 4


def _embed(idx_flat, table):
    n = idx_flat.shape[0]
    per_w = n // NUM_WORKERS
    nchunks = per_w // CHUNK
    mesh = plsc.VectorSubcoreMesh(core_axis_name="c", subcore_axis_name="s")

    @functools.partial(
        pl.kernel,
        mesh=mesh,
        out_type=jax.ShapeDtypeStruct((n, EMBED_DIM), jnp.float32),
        compiler_params=pltpu.CompilerParams(use_tc_tiling_on_sc=True),
        scratch_types=(
            [pltpu.VMEM((CHUNK,), jnp.int32) for _ in range(NBUF)]
            + [pltpu.VMEM((CHUNK, EMBED_DIM), jnp.float32)
               for _ in range(NBUF)]
            + [pltpu.SemaphoreType.DMA for _ in range(2 * NBUF)]
        ),
    )
    def k(table_hbm, idx_hbm, out_hbm, *bufs):
        idxb = bufs[:NBUF]
        rows = bufs[NBUF:2 * NBUF]
        gsem = bufs[2 * NBUF:3 * NBUF]
        ssem = bufs[3 * NBUF:4 * NBUF]
        wid = lax.axis_index("s") * 2 + lax.axis_index("c")
        base = wid * per_w

        def load_idx(c, b):
            pltpu.sync_copy(idx_hbm.at[pl.ds(base + c * CHUNK, CHUNK)],
                            idxb[b])

        def gather(b):
            return pltpu.make_async_copy(
                table_hbm.at[idxb[b]], rows[b], gsem[b])

        def store(c, b):
            return pltpu.make_async_copy(
                rows[b], out_hbm.at[pl.ds(base + c * CHUNK, CHUNK)], ssem[b])

        for b in range(NBUF):
            load_idx(b, b)
            gather(b).start()

        @pl.loop(0, nchunks - NBUF, step=NBUF)
        def _(g):
            for b in range(NBUF):
                c = g + b
                gather(b).wait()
                store(c, b).start()
                load_idx(c + NBUF, b)
                store(c, b).wait()
                gather(b).start()

        for b in range(NBUF):
            c = nchunks - NBUF + b
            gather(b).wait()
            store(c, b).start()
        for b in range(NBUF):
            store(nchunks - NBUF + b, b).wait()

    return k(table, idx_flat)


def kernel(x, table):
    b, s = x.shape
    out = _embed(x.T.reshape(-1), table)
    return out.reshape(s, b, EMBED_DIM).transpose(1, 0, 2)
